# baseline (device time: 149273 ns/iter reference)
import jax
import jax.numpy as jnp
from jax import lax
from jax.experimental import pallas as pl
from jax.experimental.pallas import tpu as pltpu

N_DEV = 8
B, S, D = 4, 256, 4096
M = B * S
DC = 1024
DCL = DC // N_DEV
H, Dh, Dr = 32, 128, 64
HL = H // N_DEV
KW = HL * Dh
QW = HL * Dr
SL = M // N_DEV
SCALE = (Dh + Dr) ** -0.5

BF = jnp.bfloat16
F32 = jnp.float32

import os
_DIAG_SKIP_AG = bool(int(os.environ.get("DIAG_SKIP_AG", "0")))


def _body(x_ref, wdkv_ref, wuk_ref, wuv_ref, wkr_ref, wq_hbm, wqr_hbm, wo_ref,
          out_ref,
          c_all, wukc, wuvc, obuf, orows, wo_stage, wq_st, wqr_st,
          a2a_send, a2a_recv, ao_send, ao_recv,
          p1_send, p1_recv, p2_send, p2_recv, p3_send, p3_recv,
          wo_sem, wq_sem):
    my = lax.axis_index("i")

    def dot(a, b):
        return lax.dot_general(a, b, (((a.ndim - 1,), (0,)), ((), ())),
                               preferred_element_type=F32)

    def dot_t(a, b):
        return lax.dot_general(a, b, (((1,), (1,)), ((), ())),
                               preferred_element_type=F32)

    DH = D // 2

    def wo_dma(qi):
        jd = lax.rem(my + qi // 2, N_DEV)
        return pltpu.make_async_copy(
            wo_ref.at[pl.ds(jd * KW, KW), pl.ds((qi % 2) * DH, DH)],
            wo_stage.at[qi % 2],
            wo_sem.at[qi % 2],
        )

    started = []

    wq_dma = pltpu.make_async_copy(
        wq_hbm.at[:, pl.ds(my * KW, KW)], wq_st, wq_sem.at[0])
    wqr_dma = pltpu.make_async_copy(
        wqr_hbm.at[:, pl.ds(my * QW, QW)], wqr_st, wq_sem.at[1])
    wq_dma.start()
    wqr_dma.start()
    wo_dma(0).start()
    wo_dma(1).start()

    for d in range(1, N_DEV):
        p = lax.rem(my + d, N_DEV)
        for t, src, dstbuf in ((0, wuk_ref, wukc), (1, wuv_ref, wuvc)):
            r = pltpu.make_async_remote_copy(
                src_ref=src.at[:, pl.ds(p * KW, KW)],
                dst_ref=dstbuf.at[pl.ds(my * DCL, DCL), :],
                send_sem=a2a_send.at[t, p],
                recv_sem=a2a_recv.at[t, my],
                device_id=(p,),
                device_id_type=pl.DeviceIdType.MESH,
            )
            r.start()
            started.append(r)

    xv = x_ref[:]
    c_all[pl.ds(my * M, M), :] = dot(xv, wdkv_ref[:]).astype(BF)
    for d in range(1, N_DEV):
        p = lax.rem(my + d, N_DEV)
        r = pltpu.make_async_remote_copy(
            src_ref=c_all.at[pl.ds(my * M, M), :],
            dst_ref=c_all.at[pl.ds(my * M, M), :],
            send_sem=a2a_send.at[2, p],
            recv_sem=a2a_recv.at[2, my],
            device_id=(p,),
            device_id_type=pl.DeviceIdType.MESH,
        )
        r.start()
        started.append(r)

    wukc[pl.ds(my * DCL, DCL), :] = wuk_ref[:, pl.ds(my * KW, KW)]
    wuvc[pl.ds(my * DCL, DCL), :] = wuv_ref[:, pl.ds(my * KW, KW)]

    kr_all = dot(xv, wkr_ref[:]).astype(BF)
    wq_dma.wait()
    q_all = dot(xv, wq_st[:].astype(BF)).astype(BF)
    wqr_dma.wait()
    qr_all = dot(xv, wqr_st[:].astype(BF)).astype(BF)

    for d in range(1, N_DEV):
        s = lax.rem(my + d, N_DEV)
        for t, dstbuf in ((0, wukc), (1, wuvc)):
            r = pltpu.make_async_remote_copy(
                src_ref=dstbuf.at[pl.ds(s * DCL, DCL), :],
                dst_ref=dstbuf.at[pl.ds(s * DCL, DCL), :],
                send_sem=a2a_send.at[t, s],
                recv_sem=a2a_recv.at[t, s],
                device_id=(s,),
                device_id_type=pl.DeviceIdType.MESH,
            )
            r.wait_recv()
        r = pltpu.make_async_remote_copy(
            src_ref=c_all.at[pl.ds(s * M, M), :],
            dst_ref=c_all.at[pl.ds(s * M, M), :],
            send_sem=a2a_send.at[2, s],
            recv_sem=a2a_recv.at[2, s],
            device_id=(s,),
            device_id_type=pl.DeviceIdType.MESH,
        )
        r.wait_recv()

    k_acc = jnp.zeros((M, KW), F32)
    v_acc = jnp.zeros((M, KW), F32)
    for i in range(N_DEV):
        ci = c_all[i * M:(i + 1) * M, :]
        wi = slice(i * DCL, (i + 1) * DCL)
        k_acc = k_acc + dot(ci, wukc[wi, :])
        v_acc = v_acc + dot(ci, wuvc[wi, :])
    k_all = k_acc.astype(BF)
    v_all = v_acc.astype(BF)

    for b in range(B):
        r0 = b * S
        krb = kr_all[r0:r0 + S, :]
        for h in range(HL):
            c0 = h * Dh
            q = q_all[r0:r0 + S, c0:c0 + Dh]
            k = k_all[r0:r0 + S, c0:c0 + Dh]
            v = v_all[r0:r0 + S, c0:c0 + Dh]
            qr = qr_all[r0:r0 + S, h * Dr:(h + 1) * Dr]
            sc = (dot_t(q, k) + dot_t(qr, krb)) * SCALE
            mx = jnp.max(sc, axis=1, keepdims=True)
            pr = jnp.exp(sc - mx)
            pr = pr / jnp.sum(pr, axis=1, keepdims=True)
            o = dot(pr.astype(BF), v)
            obuf[r0:r0 + S, c0:c0 + Dh] = o.astype(BF)
        for p in (2 * b, 2 * b + 1):
            @pl.when(p != my)
            def _send(p=p):
                r = pltpu.make_async_remote_copy(
                    src_ref=obuf.at[pl.ds(p * SL, SL), :],
                    dst_ref=orows.at[pl.ds(my * SL, SL), :],
                    send_sem=ao_send.at[p],
                    recv_sem=ao_recv.at[my],
                    device_id=(p,),
                    device_id_type=pl.DeviceIdType.MESH,
                )
                r.start()

            @pl.when(p == my)
            def _copy(p=p):
                orows[p * SL:(p + 1) * SL, :] = obuf[p * SL:(p + 1) * SL, :]

    acc = [jnp.zeros((SL, DH), F32), jnp.zeros((SL, DH), F32)]
    for d in range(N_DEV):
        jd = lax.rem(my + d, N_DEV)
        if d > 0:
            r = pltpu.make_async_remote_copy(
                src_ref=orows.at[pl.ds(jd * SL, SL), :],
                dst_ref=orows.at[pl.ds(jd * SL, SL), :],
                send_sem=ao_send.at[jd],
                recv_sem=ao_recv.at[jd],
                device_id=(jd,),
                device_id_type=pl.DeviceIdType.MESH,
            )
            r.wait_recv()
        ob_j = orows[pl.ds(jd * SL, SL), :]
        for half in (0, 1):
            qi = 2 * d + half
            wo_dma(qi).wait()
            acc[half] = acc[half] + dot(ob_j, wo_stage[half].astype(BF))
            if qi + 2 < 2 * N_DEV:
                wo_dma(qi + 2).start()
    out_ref[pl.ds(my * SL, SL), 0:DH] = acc[0].astype(BF)
    out_ref[pl.ds(my * SL, SL), DH:D] = acc[1].astype(BF)

    if not _DIAG_SKIP_AG:
        nbr = [my ^ 1, my ^ 3, my ^ 4]

        def chunk_ref(cid, half=None):
            if half is None:
                return out_ref.at[pl.ds(cid * SL, SL), :]
            return out_ref.at[pl.ds(cid * SL, SL), pl.ds(half * DH, DH)]

        def ag_rdma(src_cid, dst_cid, send_sem, recv_sem, dev, half=None):
            return pltpu.make_async_remote_copy(
                src_ref=chunk_ref(src_cid, half),
                dst_ref=chunk_ref(dst_cid, half),
                send_sem=send_sem,
                recv_sem=recv_sem,
                device_id=(dev,),
                device_id_type=pl.DeviceIdType.MESH,
            )

        for l in range(3):
            r = ag_rdma(my, my, p1_send.at[l], p1_recv.at[l], nbr[l])
            r.start()
            started.append(r)
        relay_src = [my ^ 3, my ^ 4, my ^ 1]
        need_recv = [1, 2, 0]
        for l in range(3):
            j = need_recv[l]
            ag_rdma(nbr[j], nbr[j], p1_send.at[j], p1_recv.at[j],
                    nbr[j]).wait_recv()
            r = ag_rdma(relay_src[l], relay_src[l],
                        p2_send.at[l], p2_recv.at[l], nbr[l])
            r.start()
            started.append(r)
        p2_chunk = [my ^ 2, my ^ 7, my ^ 5]
        for l in range(3):
            ag_rdma(p2_chunk[l], p2_chunk[l], p2_send.at[l], p2_recv.at[l],
                    nbr[l]).wait_recv()
        for l, src_cid in ((0, my ^ 7), (1, my ^ 5)):
            r = ag_rdma(src_cid, src_cid, p3_send.at[l], p3_recv.at[l],
                        nbr[l], half=l)
            r.start()
            started.append(r)
        for l in range(2):
            ag_rdma(my ^ 6, my ^ 6, p3_send.at[l], p3_recv.at[l],
                    nbr[l], half=l).wait_recv()

    for r in started:
        r.wait_send()
    for p in range(N_DEV):
        @pl.when(p != my)
        def _waits(p=p):
            r = pltpu.make_async_remote_copy(
                src_ref=obuf.at[pl.ds(p * SL, SL), :],
                dst_ref=orows.at[pl.ds(my * SL, SL), :],
                send_sem=ao_send.at[p],
                recv_sem=ao_recv.at[my],
                device_id=(p,),
                device_id_type=pl.DeviceIdType.MESH,
            )
            r.wait_send()


def kernel(x, Wdkv, Wuk, Wuv, Wq, Wqr, Wkr, Wo):
    xb = x.reshape(M, D).astype(BF)
    wdkv = Wdkv.astype(BF)
    wuk = Wuk.astype(BF)
    wuv = Wuv.astype(BF)
    wkr = Wkr.astype(BF)

    out = pl.pallas_call(
        _body,
        out_shape=jax.ShapeDtypeStruct((M, D), BF),
        in_specs=[pl.BlockSpec(memory_space=pltpu.VMEM)] * 5
        + [pl.BlockSpec(memory_space=pl.ANY)] * 3,
        out_specs=pl.BlockSpec(memory_space=pltpu.VMEM),
        scratch_shapes=[
            pltpu.VMEM((N_DEV * M, DCL), BF),
            pltpu.VMEM((DC, KW), BF),
            pltpu.VMEM((DC, KW), BF),
            pltpu.VMEM((M, KW), BF),
            pltpu.VMEM((M, KW), BF),
            pltpu.VMEM((2, KW, D // 2), F32),
            pltpu.VMEM((D, KW), F32),
            pltpu.VMEM((D, QW), F32),
            pltpu.SemaphoreType.DMA((3, N_DEV)),
            pltpu.SemaphoreType.DMA((3, N_DEV)),
            pltpu.SemaphoreType.DMA((N_DEV,)),
            pltpu.SemaphoreType.DMA((N_DEV,)),
            pltpu.SemaphoreType.DMA((3,)),
            pltpu.SemaphoreType.DMA((3,)),
            pltpu.SemaphoreType.DMA((3,)),
            pltpu.SemaphoreType.DMA((3,)),
            pltpu.SemaphoreType.DMA((2,)),
            pltpu.SemaphoreType.DMA((2,)),
            pltpu.SemaphoreType.DMA((2,)),
            pltpu.SemaphoreType.DMA((2,)),
        ],
        compiler_params=pltpu.CompilerParams(
            vmem_limit_bytes=64 * 1024 * 1024,
        ),
    )(xb, wdkv, wuk, wuv, wkr, Wq, Wqr, Wo)
    return out.reshape(B, S, D).astype(jnp.float32)


# device time: 92950 ns/iter; 1.6059x vs baseline; 1.6059x over previous
import jax
import jax.numpy as jnp
from jax import lax
from jax.experimental import pallas as pl
from jax.experimental.pallas import tpu as pltpu

N_DEV = 8
B, S, D = 4, 256, 4096
M = B * S
DC = 1024
DCL = DC // N_DEV
H, Dh, Dr = 32, 128, 64
HL = H // N_DEV
KW = HL * Dh
QW = HL * Dr
SL = M // N_DEV
SCALE = (Dh + Dr) ** -0.5

BF = jnp.bfloat16
F32 = jnp.float32

import os
_DIAG_SKIP_AG = bool(int(os.environ.get("DIAG_SKIP_AG", "0")))
_DIAG_SKIP_COMM = bool(int(os.environ.get("DIAG_SKIP_COMM", "0")))
_DIAG_SKIP_AG = _DIAG_SKIP_AG or _DIAG_SKIP_COMM


def _body(x_ref, wdkv_ref, wuk_ref, wuv_ref, wkr_ref, wq_hbm, wqr_hbm, wo_ref,
          out_ref,
          c_all, wukc, wuvc, obuf, orows, wo_stage, wq_st, wqr_st,
          a2a_send, a2a_recv, ao_send, ao_recv,
          p1_send, p1_recv, p2_send, p2_recv, p3_send, p3_recv,
          wo_sem, wq_sem):
    my = lax.axis_index("i")

    def dot(a, b):
        return lax.dot_general(a, b, (((a.ndim - 1,), (0,)), ((), ())),
                               preferred_element_type=F32)

    def dot_t(a, b):
        return lax.dot_general(a, b, (((1,), (1,)), ((), ())),
                               preferred_element_type=F32)

    DH = D // 2

    def wo_dma(qi):
        jd = lax.rem(my + qi // 2, N_DEV)
        return pltpu.make_async_copy(
            wo_ref.at[pl.ds(jd * KW, KW), pl.ds((qi % 2) * DH, DH)],
            wo_stage.at[qi % 2],
            wo_sem.at[qi % 2],
        )

    started = []

    wq_dma = pltpu.make_async_copy(
        wq_hbm.at[:, pl.ds(my * KW, KW)], wq_st, wq_sem.at[0])
    wqr_dma = pltpu.make_async_copy(
        wqr_hbm.at[:, pl.ds(my * QW, QW)], wqr_st, wq_sem.at[1])
    wq_dma.start()
    wqr_dma.start()
    wo_dma(0).start()
    wo_dma(1).start()

    for d in range(1, 0 if _DIAG_SKIP_COMM else N_DEV):
        p = lax.rem(my + d, N_DEV)
        for t, src, dstbuf in ((0, wuk_ref, wukc), (1, wuv_ref, wuvc)):
            r = pltpu.make_async_remote_copy(
                src_ref=src.at[:, pl.ds(p * KW, KW)],
                dst_ref=dstbuf.at[pl.ds(my * DCL, DCL), :],
                send_sem=a2a_send.at[t, p],
                recv_sem=a2a_recv.at[t, my],
                device_id=(p,),
                device_id_type=pl.DeviceIdType.MESH,
            )
            r.start()
            started.append(r)

    xv = x_ref[:]
    c_all[pl.ds(my * M, M), :] = dot(xv, wdkv_ref[:]).astype(BF)
    for d in range(1, 0 if _DIAG_SKIP_COMM else N_DEV):
        p = lax.rem(my + d, N_DEV)
        r = pltpu.make_async_remote_copy(
            src_ref=c_all.at[pl.ds(my * M, M), :],
            dst_ref=c_all.at[pl.ds(my * M, M), :],
            send_sem=a2a_send.at[2, p],
            recv_sem=a2a_recv.at[2, my],
            device_id=(p,),
            device_id_type=pl.DeviceIdType.MESH,
        )
        r.start()
        started.append(r)

    wukc[pl.ds(my * DCL, DCL), :] = wuk_ref[:, pl.ds(my * KW, KW)]
    wuvc[pl.ds(my * DCL, DCL), :] = wuv_ref[:, pl.ds(my * KW, KW)]

    kr_all = dot(xv, wkr_ref[:]).astype(BF)
    wq_dma.wait()
    q_all = dot(xv, wq_st[:].astype(BF)).astype(BF)
    wqr_dma.wait()
    qr_all = dot(xv, wqr_st[:].astype(BF)).astype(BF)

    for d in range(1, 0 if _DIAG_SKIP_COMM else N_DEV):
        s = lax.rem(my + d, N_DEV)
        for t, dstbuf in ((0, wukc), (1, wuvc)):
            r = pltpu.make_async_remote_copy(
                src_ref=dstbuf.at[pl.ds(s * DCL, DCL), :],
                dst_ref=dstbuf.at[pl.ds(s * DCL, DCL), :],
                send_sem=a2a_send.at[t, s],
                recv_sem=a2a_recv.at[t, s],
                device_id=(s,),
                device_id_type=pl.DeviceIdType.MESH,
            )
            r.wait_recv()
        r = pltpu.make_async_remote_copy(
            src_ref=c_all.at[pl.ds(s * M, M), :],
            dst_ref=c_all.at[pl.ds(s * M, M), :],
            send_sem=a2a_send.at[2, s],
            recv_sem=a2a_recv.at[2, s],
            device_id=(s,),
            device_id_type=pl.DeviceIdType.MESH,
        )
        r.wait_recv()

    k_acc = jnp.zeros((M, KW), F32)
    v_acc = jnp.zeros((M, KW), F32)
    for i in range(N_DEV):
        ci = c_all[i * M:(i + 1) * M, :]
        wi = slice(i * DCL, (i + 1) * DCL)
        k_acc = k_acc + dot(ci, wukc[wi, :])
        v_acc = v_acc + dot(ci, wuvc[wi, :])
    k_all = k_acc.astype(BF)
    v_all = v_acc.astype(BF)

    for b in range(B):
        r0 = b * S
        krb = kr_all[r0:r0 + S, :]
        for h in range(HL):
            c0 = h * Dh
            q = q_all[r0:r0 + S, c0:c0 + Dh]
            k = k_all[r0:r0 + S, c0:c0 + Dh]
            v = v_all[r0:r0 + S, c0:c0 + Dh]
            qr = qr_all[r0:r0 + S, h * Dr:(h + 1) * Dr]
            sc = (dot_t(q, k) + dot_t(qr, krb)) * SCALE
            mx = jnp.max(sc, axis=1, keepdims=True)
            pr = jnp.exp(sc - mx)
            pr = pr / jnp.sum(pr, axis=1, keepdims=True)
            o = dot(pr.astype(BF), v)
            obuf[r0:r0 + S, c0:c0 + Dh] = o.astype(BF)
        for p in (2 * b, 2 * b + 1) if not _DIAG_SKIP_COMM else ():
            @pl.when(p != my)
            def _send(p=p):
                r = pltpu.make_async_remote_copy(
                    src_ref=obuf.at[pl.ds(p * SL, SL), :],
                    dst_ref=orows.at[pl.ds(my * SL, SL), :],
                    send_sem=ao_send.at[p],
                    recv_sem=ao_recv.at[my],
                    device_id=(p,),
                    device_id_type=pl.DeviceIdType.MESH,
                )
                r.start()

            @pl.when(p == my)
            def _copy(p=p):
                orows[p * SL:(p + 1) * SL, :] = obuf[p * SL:(p + 1) * SL, :]

    acc = [jnp.zeros((SL, DH), F32), jnp.zeros((SL, DH), F32)]
    for d in range(N_DEV):
        jd = lax.rem(my + d, N_DEV)
        if d > 0 and not _DIAG_SKIP_COMM:
            r = pltpu.make_async_remote_copy(
                src_ref=orows.at[pl.ds(jd * SL, SL), :],
                dst_ref=orows.at[pl.ds(jd * SL, SL), :],
                send_sem=ao_send.at[jd],
                recv_sem=ao_recv.at[jd],
                device_id=(jd,),
                device_id_type=pl.DeviceIdType.MESH,
            )
            r.wait_recv()
        ob_j = orows[pl.ds(jd * SL, SL), :]
        for half in (0, 1):
            qi = 2 * d + half
            wo_dma(qi).wait()
            acc[half] = acc[half] + dot(ob_j, wo_stage[half].astype(BF))
            if qi + 2 < 2 * N_DEV:
                wo_dma(qi + 2).start()
    out_ref[pl.ds(my * SL, SL), 0:DH] = acc[0].astype(BF)
    out_ref[pl.ds(my * SL, SL), DH:D] = acc[1].astype(BF)

    if not _DIAG_SKIP_AG:
        nbr = [my ^ 1, my ^ 3, my ^ 4]

        def chunk_ref(cid, half=None):
            if half is None:
                return out_ref.at[pl.ds(cid * SL, SL), :]
            return out_ref.at[pl.ds(cid * SL, SL), pl.ds(half * DH, DH)]

        def ag_rdma(src_cid, dst_cid, send_sem, recv_sem, dev, half=None):
            return pltpu.make_async_remote_copy(
                src_ref=chunk_ref(src_cid, half),
                dst_ref=chunk_ref(dst_cid, half),
                send_sem=send_sem,
                recv_sem=recv_sem,
                device_id=(dev,),
                device_id_type=pl.DeviceIdType.MESH,
            )

        for l in range(3):
            r = ag_rdma(my, my, p1_send.at[l], p1_recv.at[l], nbr[l])
            r.start()
            started.append(r)
        relay_src = [my ^ 3, my ^ 4, my ^ 1]
        need_recv = [1, 2, 0]
        for l in range(3):
            j = need_recv[l]
            ag_rdma(nbr[j], nbr[j], p1_send.at[j], p1_recv.at[j],
                    nbr[j]).wait_recv()
            r = ag_rdma(relay_src[l], relay_src[l],
                        p2_send.at[l], p2_recv.at[l], nbr[l])
            r.start()
            started.append(r)
        p2_chunk = [my ^ 2, my ^ 7, my ^ 5]
        for l in range(3):
            ag_rdma(p2_chunk[l], p2_chunk[l], p2_send.at[l], p2_recv.at[l],
                    nbr[l]).wait_recv()
        for l, src_cid in ((0, my ^ 7), (1, my ^ 5)):
            r = ag_rdma(src_cid, src_cid, p3_send.at[l], p3_recv.at[l],
                        nbr[l], half=l)
            r.start()
            started.append(r)
        for l in range(2):
            ag_rdma(my ^ 6, my ^ 6, p3_send.at[l], p3_recv.at[l],
                    nbr[l], half=l).wait_recv()

    for r in started:
        r.wait_send()
    for p in range(0 if _DIAG_SKIP_COMM else N_DEV):
        @pl.when(p != my)
        def _waits(p=p):
            r = pltpu.make_async_remote_copy(
                src_ref=obuf.at[pl.ds(p * SL, SL), :],
                dst_ref=orows.at[pl.ds(my * SL, SL), :],
                send_sem=ao_send.at[p],
                recv_sem=ao_recv.at[my],
                device_id=(p,),
                device_id_type=pl.DeviceIdType.MESH,
            )
            r.wait_send()


def kernel(x, Wdkv, Wuk, Wuv, Wq, Wqr, Wkr, Wo):
    xb = x.reshape(M, D).astype(BF)
    wdkv = Wdkv.astype(BF)
    wuk = Wuk.astype(BF)
    wuv = Wuv.astype(BF)
    wkr = Wkr.astype(BF)

    out = pl.pallas_call(
        _body,
        out_shape=jax.ShapeDtypeStruct((M, D), BF),
        in_specs=[pl.BlockSpec(memory_space=pltpu.VMEM)] * 5
        + [pl.BlockSpec(memory_space=pl.ANY)] * 3,
        out_specs=pl.BlockSpec(memory_space=pltpu.VMEM),
        scratch_shapes=[
            pltpu.VMEM((N_DEV * M, DCL), BF),
            pltpu.VMEM((DC, KW), BF),
            pltpu.VMEM((DC, KW), BF),
            pltpu.VMEM((M, KW), BF),
            pltpu.VMEM((M, KW), BF),
            pltpu.VMEM((2, KW, D // 2), F32),
            pltpu.VMEM((D, KW), F32),
            pltpu.VMEM((D, QW), F32),
            pltpu.SemaphoreType.DMA((3, N_DEV)),
            pltpu.SemaphoreType.DMA((3, N_DEV)),
            pltpu.SemaphoreType.DMA((N_DEV,)),
            pltpu.SemaphoreType.DMA((N_DEV,)),
            pltpu.SemaphoreType.DMA((3,)),
            pltpu.SemaphoreType.DMA((3,)),
            pltpu.SemaphoreType.DMA((3,)),
            pltpu.SemaphoreType.DMA((3,)),
            pltpu.SemaphoreType.DMA((2,)),
            pltpu.SemaphoreType.DMA((2,)),
            pltpu.SemaphoreType.DMA((2,)),
            pltpu.SemaphoreType.DMA((2,)),
        ],
        compiler_params=pltpu.CompilerParams(
            vmem_limit_bytes=64 * 1024 * 1024,
        ),
    )(xb, wdkv, wuk, wuv, wkr, Wq, Wqr, Wo)
    return out.reshape(B, S, D).astype(jnp.float32)
